# Initial kernel scaffold; baseline (speedup 1.0000x reference)
import functools

import jax
import jax.numpy as jnp
from jax import lax
from jax.experimental import pallas as pl
from jax.experimental.pallas import tpu as pltpu
from jax.experimental.pallas import tpu_sc as plsc

N = 10000
MP = 10240
F = 500
D = 16
E = 160000
NW = 32
CH = 128
NCH = 40
EPAD = NW * CH * NCH
RPT = MP // 16
DW = 8

_MESH = plsc.VectorSubcoreMesh(
    core_axis_name="c", subcore_axis_name="s", num_cores=2, num_subcores=16
)


@functools.partial(
    pl.kernel,
    out_type=jax.ShapeDtypeStruct((2, MP, DW), jnp.float32),
    mesh=_MESH,
    scratch_types=[
        pltpu.VMEM((NCH, CH), jnp.int32),
        pltpu.VMEM((CH, DW), jnp.float32),
        pltpu.MemorySpace.VMEM_SHARED((MP, DW), jnp.float32),
    ],
)
def _sc_deg(dst_hbm, zeros_hbm, ones_hbm, out_hbm, dst_v, ones_v, acc_sh):
    cid = lax.axis_index("c")
    sid = lax.axis_index("s")
    wid = cid * 16 + sid
    rbase = sid * RPT
    pltpu.sync_copy(zeros_hbm.at[pl.ds(rbase, RPT)], acc_sh.at[pl.ds(rbase, RPT)])
    pltpu.sync_copy(ones_hbm, ones_v)
    pltpu.sync_copy(dst_hbm.at[wid], dst_v)
    plsc.subcore_barrier()

    def body(j, carry):
        pltpu.sync_copy(ones_v, acc_sh.at[dst_v.at[j]], add=True)
        return carry

    lax.fori_loop(0, NCH, body, 0)
    plsc.subcore_barrier()
    pltpu.sync_copy(acc_sh.at[pl.ds(rbase, RPT)], out_hbm.at[cid, pl.ds(rbase, RPT)])


@functools.partial(
    pl.kernel,
    out_type=jax.ShapeDtypeStruct((2, MP, D), jnp.float32),
    mesh=_MESH,
    scratch_types=[
        pltpu.VMEM((NCH, CH), jnp.int32),
        pltpu.VMEM((NCH, CH), jnp.int32),
        pltpu.VMEM((CH, D), jnp.float32),
        pltpu.VMEM((CH, D), jnp.float32),
        pltpu.SemaphoreType.DMA,
        pltpu.SemaphoreType.DMA,
        pltpu.MemorySpace.VMEM_SHARED((MP, D), jnp.float32),
    ],
)
def _sc_agg(g_hbm, src_hbm, dst_hbm, zeros_hbm, out_hbm,
            src_v, dst_v, buf_a, buf_b, sem_a, sem_b, acc_sh):
    cid = lax.axis_index("c")
    sid = lax.axis_index("s")
    wid = cid * 16 + sid
    rbase = sid * RPT
    pltpu.sync_copy(zeros_hbm.at[pl.ds(rbase, RPT)], acc_sh.at[pl.ds(rbase, RPT)])
    pltpu.sync_copy(src_hbm.at[wid], src_v)
    pltpu.sync_copy(dst_hbm.at[wid], dst_v)
    plsc.subcore_barrier()

    def fire(j, buf, sem):
        pltpu.async_copy(g_hbm.at[src_v.at[j]], buf, sem)

    def drain(buf, sem):
        pltpu.make_async_copy(g_hbm.at[src_v.at[0]], buf, sem).wait()

    fire(0, buf_a, sem_a)

    def body(jj, carry):
        j0 = 2 * jj
        fire(j0 + 1, buf_b, sem_b)
        drain(buf_a, sem_a)
        pltpu.sync_copy(buf_a, acc_sh.at[dst_v.at[j0]], add=True)

        @pl.when(jj + 1 < NCH // 2)
        def _():
            fire(j0 + 2, buf_a, sem_a)

        drain(buf_b, sem_b)
        pltpu.sync_copy(buf_b, acc_sh.at[dst_v.at[j0 + 1]], add=True)
        return carry

    lax.fori_loop(0, NCH // 2, body, 0)
    plsc.subcore_barrier()
    pltpu.sync_copy(acc_sh.at[pl.ds(rbase, RPT)], out_hbm.at[cid, pl.ds(rbase, RPT)])


def _mm_body(x_ref, w_ref, o_ref):
    o_ref[...] = jnp.dot(x_ref[...], w_ref[...], preferred_element_type=jnp.float32)


def _tc_matmul(x, w1):
    return pl.pallas_call(
        _mm_body,
        grid=(5,),
        in_specs=[
            pl.BlockSpec((2000, F), lambda i: (i, 0)),
            pl.BlockSpec((F, D), lambda i: (0, 0)),
        ],
        out_specs=pl.BlockSpec((2000, D), lambda i: (i, 0)),
        out_shape=jax.ShapeDtypeStruct((N, D), jnp.float32),
    )(x, w1)


def _prep_body(h1_ref, dp_ref, g1_ref, dis_ref):
    deg = dp_ref[0][:, 0:1] + dp_ref[1][:, 0:1] + 1.0
    dis = lax.rsqrt(deg)
    dis_ref[...] = dis
    g1_ref[0:N, :] = h1_ref[...] * dis[0:N, :]
    g1_ref[N:MP, :] = jnp.zeros((MP - N, D), jnp.float32)


def _tc_prep(h1, deg_part):
    return pl.pallas_call(
        _prep_body,
        out_shape=(
            jax.ShapeDtypeStruct((MP, D), jnp.float32),
            jax.ShapeDtypeStruct((MP, 1), jnp.float32),
        ),
    )(h1, deg_part)


def _mid_body(a_ref, g1_ref, dis_ref, b1_ref, g2_ref):
    acc = a_ref[0] + a_ref[1] + g1_ref[...]
    t = acc * dis_ref[...] + b1_ref[...]
    r = jnp.maximum(t, 0.0)
    g2 = r * dis_ref[...]
    row = lax.broadcasted_iota(jnp.int32, (MP, D), 0)
    g2_ref[...] = jnp.where(row < N, g2, 0.0)


def _tc_mid(acc1, g1, dis, b1):
    return pl.pallas_call(
        _mid_body,
        out_shape=jax.ShapeDtypeStruct((MP, D), jnp.float32),
    )(acc1, g1, dis, b1)


def _fin_body(a_ref, g2_ref, dis_ref, w2_ref, b2_ref, o_ref):
    acc = a_ref[0] + a_ref[1] + g2_ref[...]
    t = acc * dis_ref[...]
    h = jnp.dot(t, w2_ref[...], preferred_element_type=jnp.float32) + b2_ref[...]
    col = lax.broadcasted_iota(jnp.int32, (MP, D), 1)
    mask = col < 3
    m = jnp.where(mask, h, jnp.float32(-1e30))
    mx = jnp.max(m, axis=1, keepdims=True)
    e = jnp.where(mask, jnp.exp(m - mx), 0.0)
    lse = jnp.log(jnp.sum(e, axis=1, keepdims=True))
    o_ref[...] = m - mx - lse


def _tc_final(acc2, g2, dis, w2p, b2p):
    return pl.pallas_call(
        _fin_body,
        out_shape=jax.ShapeDtypeStruct((MP, D), jnp.float32),
    )(acc2, g2, dis, w2p, b2p)


@jax.jit
def kernel(x, edge, W1, b1, W2, b2):
    e32 = edge.astype(jnp.int32)
    pad = jnp.full((EPAD - E,), N, jnp.int32)
    src = jnp.concatenate([e32[0], pad]).reshape(NW, NCH, CH)
    dst = jnp.concatenate([e32[1], pad]).reshape(NW, NCH, CH)
    zeros_d = jnp.zeros((MP, D), jnp.float32)
    zeros_w = jnp.zeros((MP, DW), jnp.float32)
    ones_w = jnp.ones((CH, DW), jnp.float32)
    b1r = b1.reshape(1, D)
    w2p = jnp.pad(W2, ((0, 0), (0, D - 3)))
    b2p = jnp.pad(b2, (0, D - 3)).reshape(1, D)

    deg_part = _sc_deg(dst, zeros_w, ones_w)
    h1 = _tc_matmul(x, W1)
    g1, dis = _tc_prep(h1, deg_part)
    acc1 = _sc_agg(g1, src, dst, zeros_d)
    g2 = _tc_mid(acc1, g1, dis, b1r)
    acc2 = _sc_agg(g2, src, dst, zeros_d)
    out16 = _tc_final(acc2, g2, dis, w2p, b2p)
    return out16[:N, :3]


# speedup vs baseline: 22.5944x; 22.5944x over previous
import functools

import jax
import jax.numpy as jnp
from jax import lax
from jax.experimental import pallas as pl
from jax.experimental.pallas import tpu as pltpu
from jax.experimental.pallas import tpu_sc as plsc

N = 10000
MP = 10240
F = 500
D = 16
E = 160000
NW = 32
CH = 128
NCH = 40
EPAD = NW * CH * NCH
RPT = MP // 16
DW = 8

_MESH = plsc.VectorSubcoreMesh(
    core_axis_name="c", subcore_axis_name="s", num_cores=2, num_subcores=16
)
_SC_PARAMS = pltpu.CompilerParams(use_tc_tiling_on_sc=False)


@functools.partial(
    pl.kernel,
    out_type=jax.ShapeDtypeStruct((2, MP, DW), jnp.float32),
    mesh=_MESH,
    scratch_types=[
        pltpu.VMEM((NCH, CH), jnp.int32),
        pltpu.VMEM((CH, DW), jnp.float32),
        pltpu.MemorySpace.VMEM_SHARED((MP, DW), jnp.float32),
    ],
    compiler_params=_SC_PARAMS,
)
def _sc_deg(dst_hbm, zeros_hbm, ones_hbm, out_hbm, dst_v, ones_v, acc_sh):
    cid = lax.axis_index("c")
    sid = lax.axis_index("s")
    wid = cid * 16 + sid
    rbase = sid * RPT
    pltpu.sync_copy(zeros_hbm.at[pl.ds(rbase, RPT)], acc_sh.at[pl.ds(rbase, RPT)])
    pltpu.sync_copy(ones_hbm, ones_v)
    pltpu.sync_copy(dst_hbm.at[wid], dst_v)
    plsc.subcore_barrier()

    def body(j, carry):
        pltpu.sync_copy(ones_v, acc_sh.at[dst_v.at[j]], add=True)
        return carry

    lax.fori_loop(0, NCH, body, 0)
    plsc.subcore_barrier()
    pltpu.sync_copy(acc_sh.at[pl.ds(rbase, RPT)], out_hbm.at[cid, pl.ds(rbase, RPT)])


@functools.partial(
    pl.kernel,
    out_type=jax.ShapeDtypeStruct((2, MP, D), jnp.float32),
    mesh=_MESH,
    scratch_types=[
        pltpu.VMEM((NCH, CH), jnp.int32),
        pltpu.VMEM((NCH, CH), jnp.int32),
        pltpu.VMEM((CH, D), jnp.float32),
        pltpu.VMEM((CH, D), jnp.float32),
        pltpu.SemaphoreType.DMA,
        pltpu.SemaphoreType.DMA,
        pltpu.MemorySpace.VMEM_SHARED((MP, D), jnp.float32),
    ],
    compiler_params=_SC_PARAMS,
)
def _sc_agg(g_hbm, src_hbm, dst_hbm, zeros_hbm, out_hbm,
            src_v, dst_v, buf_a, buf_b, sem_a, sem_b, acc_sh):
    cid = lax.axis_index("c")
    sid = lax.axis_index("s")
    wid = cid * 16 + sid
    rbase = sid * RPT
    pltpu.sync_copy(zeros_hbm.at[pl.ds(rbase, RPT)], acc_sh.at[pl.ds(rbase, RPT)])
    pltpu.sync_copy(src_hbm.at[wid], src_v)
    pltpu.sync_copy(dst_hbm.at[wid], dst_v)
    plsc.subcore_barrier()

    def fire(j, buf, sem):
        pltpu.async_copy(g_hbm.at[src_v.at[j]], buf, sem)

    def drain(buf, sem):
        pltpu.make_async_copy(g_hbm.at[src_v.at[0]], buf, sem).wait()

    fire(0, buf_a, sem_a)

    def body(jj, carry):
        j0 = 2 * jj
        fire(j0 + 1, buf_b, sem_b)
        drain(buf_a, sem_a)
        pltpu.sync_copy(buf_a, acc_sh.at[dst_v.at[j0]], add=True)

        @pl.when(jj + 1 < NCH // 2)
        def _():
            fire(j0 + 2, buf_a, sem_a)

        drain(buf_b, sem_b)
        pltpu.sync_copy(buf_b, acc_sh.at[dst_v.at[j0 + 1]], add=True)
        return carry

    lax.fori_loop(0, NCH // 2, body, 0)
    plsc.subcore_barrier()
    pltpu.sync_copy(acc_sh.at[pl.ds(rbase, RPT)], out_hbm.at[cid, pl.ds(rbase, RPT)])


def _mm_body(x_ref, w_ref, o_ref):
    o_ref[...] = jnp.dot(x_ref[...], w_ref[...], preferred_element_type=jnp.float32)


def _tc_matmul(x, w1):
    return pl.pallas_call(
        _mm_body,
        grid=(5,),
        in_specs=[
            pl.BlockSpec((2000, F), lambda i: (i, 0)),
            pl.BlockSpec((F, D), lambda i: (0, 0)),
        ],
        out_specs=pl.BlockSpec((2000, D), lambda i: (i, 0)),
        out_shape=jax.ShapeDtypeStruct((N, D), jnp.float32),
    )(x, w1)


def _prep_body(h1_ref, dp_ref, g1_ref, dis_ref):
    deg = dp_ref[0][:, 0:1] + dp_ref[1][:, 0:1] + 1.0
    dis = lax.rsqrt(deg)
    dis_ref[...] = dis
    g1_ref[0:N, :] = h1_ref[...] * dis[0:N, :]
    g1_ref[N:MP, :] = jnp.zeros((MP - N, D), jnp.float32)


def _tc_prep(h1, deg_part):
    return pl.pallas_call(
        _prep_body,
        out_shape=(
            jax.ShapeDtypeStruct((MP, D), jnp.float32),
            jax.ShapeDtypeStruct((MP, 1), jnp.float32),
        ),
    )(h1, deg_part)


def _mid_body(a_ref, g1_ref, dis_ref, b1_ref, g2_ref):
    acc = a_ref[0] + a_ref[1] + g1_ref[...]
    t = acc * dis_ref[...] + b1_ref[...]
    r = jnp.maximum(t, 0.0)
    g2 = r * dis_ref[...]
    row = lax.broadcasted_iota(jnp.int32, (MP, D), 0)
    g2_ref[...] = jnp.where(row < N, g2, 0.0)


def _tc_mid(acc1, g1, dis, b1):
    return pl.pallas_call(
        _mid_body,
        out_shape=jax.ShapeDtypeStruct((MP, D), jnp.float32),
    )(acc1, g1, dis, b1)


def _fin_body(a_ref, g2_ref, dis_ref, w2_ref, b2_ref, o_ref):
    acc = a_ref[0] + a_ref[1] + g2_ref[...]
    t = acc * dis_ref[...]
    h = jnp.dot(t, w2_ref[...], preferred_element_type=jnp.float32) + b2_ref[...]
    col = lax.broadcasted_iota(jnp.int32, (MP, D), 1)
    mask = col < 3
    m = jnp.where(mask, h, jnp.float32(-1e30))
    mx = jnp.max(m, axis=1, keepdims=True)
    e = jnp.where(mask, jnp.exp(m - mx), 0.0)
    lse = jnp.log(jnp.sum(e, axis=1, keepdims=True))
    o_ref[...] = m - mx - lse


def _tc_final(acc2, g2, dis, w2p, b2p):
    return pl.pallas_call(
        _fin_body,
        out_shape=jax.ShapeDtypeStruct((MP, D), jnp.float32),
    )(acc2, g2, dis, w2p, b2p)


@jax.jit
def kernel(x, edge, W1, b1, W2, b2):
    e32 = edge.astype(jnp.int32)
    pad = jnp.full((EPAD - E,), N, jnp.int32)
    src = jnp.concatenate([e32[0], pad]).reshape(NW, NCH, CH)
    dst = jnp.concatenate([e32[1], pad]).reshape(NW, NCH, CH)
    zeros_d = jnp.zeros((MP, D), jnp.float32)
    zeros_w = jnp.zeros((MP, DW), jnp.float32)
    ones_w = jnp.ones((CH, DW), jnp.float32)
    b1r = b1.reshape(1, D)
    w2p = jnp.pad(W2, ((0, 0), (0, D - 3)))
    b2p = jnp.pad(b2, (0, D - 3)).reshape(1, D)

    deg_part = _sc_deg(dst, zeros_w, ones_w)
    h1 = _tc_matmul(x, W1)
    g1, dis = _tc_prep(h1, deg_part)
    acc1 = _sc_agg(g1, src, dst, zeros_d)
    g2 = _tc_mid(acc1, g1, dis, b1r)
    acc2 = _sc_agg(g2, src, dst, zeros_d)
    out16 = _tc_final(acc2, g2, dis, w2p, b2p)
    return out16[:N, :3]
